# Initial kernel scaffold; baseline (speedup 1.0000x reference)
import jax
import jax.numpy as jnp
from jax.experimental import pallas as pl


def kernel(enc_inputs, sequence_length, current_input, embedding, W_ih, W_hh, b_ih, b_hh):
    raise NotImplementedError("write your pallas kernel here")


# speedup vs baseline: 10.7117x; 10.7117x over previous
import functools

import jax
import jax.numpy as jnp
from jax import lax
from jax.experimental import pallas as pl
from jax.experimental.pallas import tpu as pltpu
from jax.experimental.pallas import tpu_sc as plsc

B, L, V, D, H = 16, 512, 32000, 256, 256
T = 64
NSTEPS = L // T


@functools.lru_cache(maxsize=None)
def _make_gather():
    info = plsc.get_sparse_core_info()
    nw = info.num_cores * info.num_subcores
    n = B * L
    b_per_w = n // nw
    mesh = plsc.VectorSubcoreMesh(core_axis_name="c", subcore_axis_name="s")

    @functools.partial(
        pl.kernel,
        out_type=jax.ShapeDtypeStruct((n, D), jnp.float32),
        mesh=mesh,
        scratch_types=[
            pltpu.VMEM((b_per_w,), jnp.int32),
            pltpu.VMEM((b_per_w, D), jnp.float32),
            pltpu.SemaphoreType.DMA,
        ],
    )
    def gather(table_hbm, idx_hbm, out_hbm, idx_v, rows_v, sem):
        wid = lax.axis_index("s") * info.num_cores + lax.axis_index("c")
        base = wid * b_per_w
        pltpu.sync_copy(idx_hbm.at[pl.ds(base, b_per_w)], idx_v)
        pltpu.async_copy(table_hbm.at[idx_v], rows_v, sem).wait()
        pltpu.sync_copy(rows_v, out_hbm.at[pl.ds(base, b_per_w)])

    return gather


def _scan_body(emb_ref, wih_ref, whh_ref, bih_ref, bhh_ref, mask_ref,
               out_ref, last_ref, h_s, gi_s):
    i = pl.program_id(0)

    @pl.when(i == 0)
    def _():
        h_s[...] = jnp.zeros_like(h_s)

    gi_s[...] = (
        jnp.dot(emb_ref[...], wih_ref[...], preferred_element_type=jnp.float32)
        + bih_ref[...]
    )

    whh = whh_ref[...]
    bhh = bhh_ref[...]

    def step(t, h):
        gh = jnp.dot(h, whh, preferred_element_type=jnp.float32) + bhh
        gi = gi_s[pl.ds(t * B, B), :]
        r = jax.nn.sigmoid(gi[:, :H] + gh[:, :H])
        z = jax.nn.sigmoid(gi[:, H:2 * H] + gh[:, H:2 * H])
        n = jnp.tanh(gi[:, 2 * H:] + r * gh[:, 2 * H:])
        h_new = (1.0 - z) * n + z * h
        m = mask_ref[pl.ds(t * B, B), :]
        out_t = m * h_new
        out_ref[pl.ds(t * B, B), :] = out_t
        return m * h_new + (1.0 - m) * h

    h = lax.fori_loop(0, T, step, h_s[...])
    h_s[...] = h

    @pl.when(i == NSTEPS - 1)
    def _():
        last_ref[...] = h


_scan = pl.pallas_call(
    _scan_body,
    grid=(NSTEPS,),
    in_specs=[
        pl.BlockSpec((T * B, D), lambda i: (i, 0)),
        pl.BlockSpec((D, 3 * H), lambda i: (0, 0)),
        pl.BlockSpec((H, 3 * H), lambda i: (0, 0)),
        pl.BlockSpec((1, 3 * H), lambda i: (0, 0)),
        pl.BlockSpec((1, 3 * H), lambda i: (0, 0)),
        pl.BlockSpec((T * B, 1), lambda i: (i, 0)),
    ],
    out_specs=[
        pl.BlockSpec((T * B, H), lambda i: (i, 0)),
        pl.BlockSpec((B, H), lambda i: (0, 0)),
    ],
    out_shape=[
        jax.ShapeDtypeStruct((L * B, H), jnp.float32),
        jax.ShapeDtypeStruct((B, H), jnp.float32),
    ],
    scratch_shapes=[
        pltpu.VMEM((B, H), jnp.float32),
        pltpu.VMEM((T * B, 3 * H), jnp.float32),
    ],
)


def kernel(enc_inputs, sequence_length, current_input, embedding,
           W_ih, W_hh, b_ih, b_hh):
    del current_input
    idx = enc_inputs.astype(jnp.int32).T.reshape(-1)
    emb = _make_gather()(embedding, idx)
    mask = (
        jnp.arange(L, dtype=jnp.int32)[:, None] < sequence_length[None, :]
    ).astype(jnp.float32).reshape(L * B, 1)
    out_flat, last = _scan(
        emb, W_ih.T, W_hh.T, b_ih[None, :], b_hh[None, :], mask
    )
    out = out_flat.reshape(L, B, H).swapaxes(0, 1)
    return out, last
